# Initial kernel scaffold; baseline (speedup 1.0000x reference)
import jax
import jax.numpy as jnp
from jax.experimental import pallas as pl


def kernel(hidden_states, router_w, w_gate, w_up, w_down):
    raise NotImplementedError("write your pallas kernel here")


# speedup vs baseline: 2.4636x; 2.4636x over previous
import functools

import jax
import jax.numpy as jnp
from jax import lax
from jax.experimental import pallas as pl
from jax.experimental.pallas import tpu as pltpu
from jax.experimental.pallas import tpu_sc as plsc

T = 4096
D = 1024
F = 512
E = 16
TOPK = 2
CAP = (T * TOPK // E) * 2
S = E * CAP
TRASH = S

NC = 2
NS = 16
NW = NC * NS


def _router_body(x_ref, rw_ref, dstw_ref, dstg_ref, w0x_ref, w1x_ref):
    x = x_ref[...]
    rw = rw_ref[...]
    logits = jnp.dot(x, rw, preferred_element_type=jnp.float32)

    eidx = lax.broadcasted_iota(jnp.int32, (T, E), 1)
    m0 = jnp.max(logits, axis=1, keepdims=True)
    e0 = jnp.min(jnp.where(logits == m0, eidx, E), axis=1)
    oh0 = eidx == e0[:, None]
    l1 = jnp.where(oh0, -jnp.inf, logits)
    m1 = jnp.max(l1, axis=1, keepdims=True)
    e1 = jnp.min(jnp.where(l1 == m1, eidx, E), axis=1)
    oh1 = eidx == e1[:, None]

    r = jnp.exp(m1[:, 0] - m0[:, 0])
    w0 = 1.0 / (1.0 + r)
    w1 = r / (1.0 + r)

    ohT = oh0.astype(jnp.float32) + oh1.astype(jnp.float32)
    BT = 512
    rows = lax.broadcasted_iota(jnp.int32, (BT, BT), 0)
    cols = lax.broadcasted_iota(jnp.int32, (BT, BT), 1)
    Lst = (rows > cols).astype(jnp.float32)
    base = jnp.zeros((1, E), jnp.float32)
    chunks = []
    for i in range(T // BT):
        blk = ohT[i * BT:(i + 1) * BT, :]
        chunks.append(jnp.dot(Lst, blk, preferred_element_type=jnp.float32)
                      + base)
        base = base + jnp.sum(blk, axis=0, keepdims=True)
    C = jnp.concatenate(chunks, axis=0)

    pos0 = jnp.sum(jnp.where(oh0, C, 0.0), axis=1).astype(jnp.int32)
    pos1 = jnp.sum(jnp.where(oh1, C, 0.0), axis=1).astype(jnp.int32)

    v0 = pos0 < CAP
    v1 = pos1 < CAP
    dstw_ref[0, :] = jnp.where(v0, e0 * CAP + pos0, TRASH)
    dstw_ref[1, :] = jnp.where(v1, e1 * CAP + pos1, TRASH)
    dstg_ref[0, :] = e0 * CAP + jnp.where(v0, pos0, 0)
    dstg_ref[1, :] = e1 * CAP + jnp.where(v1, pos1, 0)
    w0m = jnp.where(v0, w0, 0.0)
    w1m = jnp.where(v1, w1, 0.0)
    w0x_ref[...] = jnp.broadcast_to(w0m[:, None], (T, E))
    w1x_ref[...] = jnp.broadcast_to(w1m[:, None], (T, E))


def _router(x, rw):
    return pl.pallas_call(
        _router_body,
        out_shape=(
            jax.ShapeDtypeStruct((2, T), jnp.int32),
            jax.ShapeDtypeStruct((2, T), jnp.int32),
            jax.ShapeDtypeStruct((T, E), jnp.float32),
            jax.ShapeDtypeStruct((T, E), jnp.float32),
        ),
    )(x, rw)


_DSUB = 64
_DSTEPS = T // NS // _DSUB


def _dispatch(x, dstw):
    mesh = plsc.VectorSubcoreMesh(core_axis_name="c", subcore_axis_name="s")

    @functools.partial(
        pl.kernel,
        out_type=jax.ShapeDtypeStruct((S + 8, D), jnp.float32),
        mesh=mesh,
        scratch_types=[
            pltpu.VMEM((_DSUB, D), jnp.float32),
        ] + [pltpu.VMEM((_DSUB,), jnp.int32) for _ in range(_DSTEPS)] + [
            pltpu.SemaphoreType.DMA,
        ],
    )
    def k(x_hbm, dstw_hbm, buf_hbm, rows_v, *rest):
        idx_vs, sem = rest[:_DSTEPS], rest[_DSTEPS]
        cid = lax.axis_index("c")
        sid = lax.axis_index("s")
        tw = sid * (T // NS)
        for c in range(_DSTEPS):
            off = tw + c * _DSUB
            pltpu.sync_copy(dstw_hbm.at[cid, pl.ds(off, _DSUB)], idx_vs[c])
            pltpu.sync_copy(x_hbm.at[pl.ds(off, _DSUB)], rows_v)
            pltpu.async_copy(rows_v, buf_hbm.at[idx_vs[c]], sem).wait()

    return k(x, dstw)


_BLKC = 512


def _mlp_body(buf_ref, wg_ref, wu_ref, wd_ref, out_ref):
    xb = buf_ref[...]
    g = jnp.dot(xb, wg_ref[0], preferred_element_type=jnp.float32)
    u = jnp.dot(xb, wu_ref[0], preferred_element_type=jnp.float32)
    h = (g * jax.nn.sigmoid(g)) * u
    out_ref[...] = jnp.dot(h, wd_ref[0], preferred_element_type=jnp.float32)


def _mlp(buf, w_gate, w_up, w_down):
    nblk = CAP // _BLKC
    return pl.pallas_call(
        _mlp_body,
        grid=(E, nblk),
        in_specs=[
            pl.BlockSpec((_BLKC, D), lambda e, c: (e * nblk + c, 0)),
            pl.BlockSpec((1, D, F), lambda e, c: (e, 0, 0)),
            pl.BlockSpec((1, D, F), lambda e, c: (e, 0, 0)),
            pl.BlockSpec((1, F, D), lambda e, c: (e, 0, 0)),
        ],
        out_specs=pl.BlockSpec((_BLKC, D), lambda e, c: (e * nblk + c, 0)),
        out_shape=jax.ShapeDtypeStruct((S, D), jnp.float32),
    )(buf, w_gate, w_up, w_down)


_CSUB = 32
_CSTEPS = T // NW // _CSUB


def _combine(out_buf, dstg, w0x, w1x):
    mesh = plsc.VectorSubcoreMesh(core_axis_name="c", subcore_axis_name="s")

    @functools.partial(
        pl.kernel,
        out_type=jax.ShapeDtypeStruct((T, D), jnp.float32),
        mesh=mesh,
        scratch_types=[
            pltpu.VMEM((_CSUB, D), jnp.float32),
            pltpu.VMEM((_CSUB, D), jnp.float32),
            pltpu.VMEM((_CSUB, D), jnp.float32),
            pltpu.VMEM((_CSUB,), jnp.int32),
            pltpu.VMEM((_CSUB,), jnp.int32),
            pltpu.VMEM((_CSUB, E), jnp.float32),
            pltpu.VMEM((_CSUB, E), jnp.float32),
            pltpu.SemaphoreType.DMA,
        ],
    )
    def k(out_hbm, dstg_hbm, w0x_hbm, w1x_hbm, y_hbm,
          a_v, b_v, y_v, i0_v, i1_v, w0_v, w1_v, sem):
        cid = lax.axis_index("c")
        sid = lax.axis_index("s")
        wid = sid * NC + cid
        tb = wid * (T // NW)
        for c in range(_CSTEPS):
            base = tb + c * _CSUB
            pltpu.sync_copy(dstg_hbm.at[0, pl.ds(base, _CSUB)], i0_v)
            pltpu.sync_copy(dstg_hbm.at[1, pl.ds(base, _CSUB)], i1_v)
            pltpu.sync_copy(w0x_hbm.at[pl.ds(base, _CSUB)], w0_v)
            pltpu.sync_copy(w1x_hbm.at[pl.ds(base, _CSUB)], w1_v)
            cp_a = pltpu.async_copy(out_hbm.at[i0_v], a_v, sem)
            cp_b = pltpu.async_copy(out_hbm.at[i1_v], b_v, sem)
            cp_a.wait()
            cp_b.wait()

            def tok(j, _):
                wa = w0_v[j, :]
                wb = w1_v[j, :]
                for v in range(D // 16):
                    sl = pl.ds(v * 16, 16)
                    y_v[j, sl] = a_v[j, sl] * wa + b_v[j, sl] * wb
                return 0

            lax.fori_loop(0, _CSUB, tok, 0)
            pltpu.sync_copy(y_v, y_hbm.at[pl.ds(base, _CSUB)])

    return k(out_buf, dstg, w0x, w1x)


def kernel(hidden_states, router_w, w_gate, w_up, w_down):
    dstw, dstg, w0x, w1x = _router(hidden_states, router_w)
    buf = _dispatch(hidden_states, dstw)
    out_buf = _mlp(buf, w_gate, w_up, w_down)
    return _combine(out_buf, dstg, w0x, w1x)
